# baseline (speedup 1.0000x reference)
import functools

import jax
import jax.numpy as jnp
from jax import lax
from jax.experimental import pallas as pl
from jax.experimental.pallas import tpu as pltpu
from jax.experimental.pallas import tpu_sc as plsc

NC = 2
NS = 16
L = 16
NW = NC * NS

B = 16384
D = 16
V = 1000000
BPW = B // NW
CH = 128
NCH = BPW // CH
NBLK = BPW // L

TPS = V // 128
TAIL = V - TPS * 128
TROW = TPS + 1
SLAB = TROW * 1024
SROWS = 2 * TROW * 8

Q1 = 16
Q2 = 16


def _k1_body(ut_hbm, it_hbm, su_hbm, si_hbm, tail_v, sem):
    wid = lax.axis_index("s") * NC + lax.axis_index("c")
    lane8 = wid % 8
    nt = jnp.where(lane8 < 4, 977, 976)
    start = jnp.where(lane8 < 4, lane8 * 977, 3908 + (lane8 - 4) * 976)

    for t in range(2):
        src = ut_hbm if t == 0 else it_hbm
        dst = su_hbm if t == 0 else si_hbm
        for s in range(2):
            gid = t * 2 + s

            @pl.when(wid // 8 == gid)
            def _(src=src, dst=dst, s=s):
                r8 = s * 8

                def body(i, carry):
                    tc = start + i
                    pltpu.async_copy(
                        src.at[pl.ds(r8, 8), pl.ds(tc * 128, 128)],
                        dst.at[pl.ds(s * (TROW * 8) + tc * 8, 8), :], sem)

                    @pl.when(i >= Q1)
                    def _():
                        pltpu.make_async_copy(
                            src.at[pl.ds(r8, 8), pl.ds(0, 128)],
                            dst.at[pl.ds(0, 8), :], sem).wait()
                    return carry

                lax.fori_loop(0, nt, body, 0)
                for _i in range(Q1):
                    pltpu.make_async_copy(
                        src.at[pl.ds(r8, 8), pl.ds(0, 128)],
                        dst.at[pl.ds(0, 8), :], sem).wait()

                @pl.when(lane8 == 7)
                def _():
                    pltpu.sync_copy(
                        src.at[pl.ds(r8, 8), pl.ds(TPS * 128, TAIL)], tail_v)
                    for r in range(8):
                        pltpu.sync_copy(
                            tail_v.at[r],
                            dst.at[(s * TROW + TPS) * 8 + r,
                                   pl.ds(0, TAIL)])


def _k2_body(su_hbm, si_hbm, iu_hbm, ii_hbm, out_hbm,
             iu_v, ii_v, ixu_v, ixi_v, gu_v, gi_v, o_v, sem):
    wid = lax.axis_index("s") * NC + lax.axis_index("c")
    base = wid * BPW

    pltpu.sync_copy(iu_hbm.at[wid], iu_v)
    pltpu.sync_copy(ii_hbm.at[wid], ii_v)

    def idxcomp(c, carry):
        for p in range(8):
            ju = iu_v[c, pl.ds(p * 16, 16)]
            ji = ii_v[c, pl.ds(p * 16, 16)]
            bu = ((ju >> 7) << 10) + (ju & 127)
            bi = ((ji >> 7) << 10) + (ji & 127)
            for d in range(D):
                cd = (d // 8) * SLAB + (d % 8) * 128
                ixu_v[d * NCH + c, pl.ds(p * 16, 16)] = bu + cd
                ixi_v[d * NCH + c, pl.ds(p * 16, 16)] = bi + cd
        return carry

    lax.fori_loop(0, NCH, idxcomp, 0)

    def fire(r, carry):
        d = r // NCH
        c = r % NCH
        pltpu.async_copy(su_hbm.at[ixu_v.at[r]],
                         gu_v.at[d, pl.ds(c * CH, CH)], sem)
        pltpu.async_copy(si_hbm.at[ixi_v.at[r]],
                         gi_v.at[d, pl.ds(c * CH, CH)], sem)

        @pl.when(r >= Q2)
        def _():
            pltpu.make_async_copy(su_hbm.at[ixu_v.at[0]],
                                  gu_v.at[0, pl.ds(0, CH)], sem).wait()
            pltpu.make_async_copy(si_hbm.at[ixi_v.at[0]],
                                  gi_v.at[0, pl.ds(0, CH)], sem).wait()
        return carry

    lax.fori_loop(0, D * NCH, fire, 0)
    for _i in range(Q2):
        pltpu.make_async_copy(su_hbm.at[ixu_v.at[0]],
                              gu_v.at[0, pl.ds(0, CH)], sem).wait()
        pltpu.make_async_copy(si_hbm.at[ixi_v.at[0]],
                              gi_v.at[0, pl.ds(0, CH)], sem).wait()

    def blk(bi, carry):
        kb = bi * L
        acc = gu_v[0, pl.ds(kb, L)] * gi_v[0, pl.ds(kb, L)]
        for d in range(1, D):
            acc = acc + gu_v[d, pl.ds(kb, L)] * gi_v[d, pl.ds(kb, L)]
        o_v[pl.ds(kb, L)] = 1.0 / (1.0 + jnp.exp(-acc))
        return carry

    lax.fori_loop(0, NBLK, blk, 0)

    pltpu.sync_copy(o_v, out_hbm.at[pl.ds(base, BPW)])


def kernel(idx_u, idx_i, embeds_u, embeds_i):
    mesh = plsc.VectorSubcoreMesh(
        core_axis_name="c", subcore_axis_name="s",
        num_cores=NC, num_subcores=NS)

    k1 = functools.partial(
        pl.kernel,
        out_type=(jax.ShapeDtypeStruct((SROWS, 128), jnp.float32),
                  jax.ShapeDtypeStruct((SROWS, 128), jnp.float32)),
        mesh=mesh,
        compiler_params=pltpu.CompilerParams(use_tc_tiling_on_sc=True),
        scratch_types=[pltpu.VMEM((8, TAIL), jnp.float32),
                       pltpu.SemaphoreType.DMA],
    )(_k1_body)

    k2 = functools.partial(
        pl.kernel,
        out_type=jax.ShapeDtypeStruct((B,), jnp.float32),
        mesh=mesh,
        compiler_params=pltpu.CompilerParams(use_tc_tiling_on_sc=False),
        scratch_types=[
            pltpu.VMEM((NCH, CH), jnp.int32),
            pltpu.VMEM((NCH, CH), jnp.int32),
            pltpu.VMEM((D * NCH, CH), jnp.int32),
            pltpu.VMEM((D * NCH, CH), jnp.int32),
            pltpu.VMEM((D, BPW), jnp.float32),
            pltpu.VMEM((D, BPW), jnp.float32),
            pltpu.VMEM((BPW,), jnp.float32),
            pltpu.SemaphoreType.DMA,
        ],
    )(_k2_body)

    su2, si2 = k1(embeds_u.T, embeds_i.T)
    su = su2.reshape(SROWS * 128)
    si = si2.reshape(SROWS * 128)
    iu3 = idx_u.reshape(NW, NCH, CH)
    ii3 = idx_i.reshape(NW, NCH, CH)
    return k2(su, si, iu3, ii3)


# speedup vs baseline: 19.6373x; 19.6373x over previous
import functools

import jax
import jax.numpy as jnp
from jax import lax
from jax.experimental import pallas as pl
from jax.experimental.pallas import tpu as pltpu
from jax.experimental.pallas import tpu_sc as plsc

NC = 2
NS = 16
L = 16
NW = NC * NS

B = 16384
D = 16
V = 1000000
BPW = B // NW
CH = 128
NCH = BPW // CH
NBLK = BPW // L

VP = 1000064
TAIL_OFF = 7812 * 128
TAIL = V - TAIL_OFF

WT = 16
WIN = WT * 128
NWIN = 61
Q2 = 16


def _k1_body(ut_hbm, it_hbm, su_hbm, si_hbm, b0, b1, tail_v, s0, s1, s2, s3):
    wid = lax.axis_index("s") * NC + lax.axis_index("c")
    lane8 = wid % 8
    bufs = (b0, b1)
    semr = (s0, s1)
    semw = (s2, s3)
    start = jnp.where(lane8 < 4, lane8 * 977, 3908 + (lane8 - 4) * 976)

    def window(w, carry):
        for t in range(2):
            src = ut_hbm if t == 0 else it_hbm
            dst = su_hbm if t == 0 else si_hbm
            for s in range(2):
                gid = t * 2 + s
                r8 = s * 8
                for p in range(2):
                    @pl.when((wid // 8 == gid) & (w % 2 == p))
                    def _(src=src, dst=dst, r8=r8, p=p):
                        buf = bufs[p]
                        @pl.when(w >= 2)
                        def _():
                            for rr in range(8):
                                pltpu.make_async_copy(
                                    buf.at[rr, pl.ds(0, WIN)],
                                    dst.at[pl.ds(0, WIN)],
                                    semw[p]).wait()
                        col = (start + w * WT) * 128
                        pltpu.async_copy(
                            src.at[pl.ds(r8, 8), pl.ds(col, WIN)],
                            buf, semr[p]).wait()
                        for rr in range(8):
                            pltpu.async_copy(
                                buf.at[rr, pl.ds(0, WIN)],
                                dst.at[pl.ds((r8 + rr) * VP + col, WIN)],
                                semw[p])
        return carry

    lax.fori_loop(0, NWIN, window, 0)

    for t in range(2):
        dst = su_hbm if t == 0 else si_hbm
        for s in range(2):
            gid = t * 2 + s

            @pl.when(wid // 8 == gid)
            def _(dst=dst):
                for p in range(2):
                    for rr in range(8):
                        pltpu.make_async_copy(
                            bufs[p].at[rr, pl.ds(0, WIN)],
                            dst.at[pl.ds(0, WIN)], semw[p]).wait()

    for t in range(2):
        src = ut_hbm if t == 0 else it_hbm
        dst = su_hbm if t == 0 else si_hbm
        for s in range(2):
            gid = t * 2 + s
            r8 = s * 8

            @pl.when((wid // 8 == gid) & (lane8 < 4))
            def _(src=src, dst=dst, r8=r8):
                col = (start + 976) * 128
                pltpu.async_copy(
                    src.at[pl.ds(r8, 8), pl.ds(col, 128)],
                    b0.at[:, pl.ds(0, 128)], semr[0]).wait()
                for rr in range(8):
                    pltpu.sync_copy(
                        b0.at[rr, pl.ds(0, 128)],
                        dst.at[pl.ds((r8 + rr) * VP + col, 128)])

            @pl.when((wid // 8 == gid) & (lane8 == 7))
            def _(src=src, dst=dst, r8=r8):
                pltpu.sync_copy(
                    src.at[pl.ds(r8, 8), pl.ds(TAIL_OFF, TAIL)], tail_v)
                for rr in range(8):
                    pltpu.sync_copy(
                        tail_v.at[rr],
                        dst.at[pl.ds((r8 + rr) * VP + TAIL_OFF, TAIL)])


def _k2_body(su_hbm, si_hbm, iu_hbm, ii_hbm, out_hbm,
             iu_v, ii_v, ixu_v, ixi_v, gu_v, gi_v, o_v, sem):
    wid = lax.axis_index("s") * NC + lax.axis_index("c")
    base = wid * BPW

    pltpu.sync_copy(iu_hbm.at[wid], iu_v)
    pltpu.sync_copy(ii_hbm.at[wid], ii_v)

    def idxcomp(c, carry):
        for p in range(8):
            ju = iu_v[c, pl.ds(p * 16, 16)]
            ji = ii_v[c, pl.ds(p * 16, 16)]
            for d in range(D):
                ixu_v[d * NCH + c, pl.ds(p * 16, 16)] = ju + d * VP
                ixi_v[d * NCH + c, pl.ds(p * 16, 16)] = ji + d * VP
        return carry

    lax.fori_loop(0, NCH, idxcomp, 0)

    def fire(r, carry):
        d = r // NCH
        c = r % NCH
        pltpu.async_copy(su_hbm.at[ixu_v.at[r]],
                         gu_v.at[d, pl.ds(c * CH, CH)], sem)
        pltpu.async_copy(si_hbm.at[ixi_v.at[r]],
                         gi_v.at[d, pl.ds(c * CH, CH)], sem)

        @pl.when(r >= Q2)
        def _():
            pltpu.make_async_copy(su_hbm.at[ixu_v.at[0]],
                                  gu_v.at[0, pl.ds(0, CH)], sem).wait()
            pltpu.make_async_copy(si_hbm.at[ixi_v.at[0]],
                                  gi_v.at[0, pl.ds(0, CH)], sem).wait()
        return carry

    lax.fori_loop(0, D * NCH, fire, 0)
    for _i in range(Q2):
        pltpu.make_async_copy(su_hbm.at[ixu_v.at[0]],
                              gu_v.at[0, pl.ds(0, CH)], sem).wait()
        pltpu.make_async_copy(si_hbm.at[ixi_v.at[0]],
                              gi_v.at[0, pl.ds(0, CH)], sem).wait()

    def blk(bi, carry):
        kb = bi * L
        acc = gu_v[0, pl.ds(kb, L)] * gi_v[0, pl.ds(kb, L)]
        for d in range(1, D):
            acc = acc + gu_v[d, pl.ds(kb, L)] * gi_v[d, pl.ds(kb, L)]
        o_v[pl.ds(kb, L)] = 1.0 / (1.0 + jnp.exp(-acc))
        return carry

    lax.fori_loop(0, NBLK, blk, 0)

    pltpu.sync_copy(o_v, out_hbm.at[pl.ds(base, BPW)])


def kernel(idx_u, idx_i, embeds_u, embeds_i):
    mesh = plsc.VectorSubcoreMesh(
        core_axis_name="c", subcore_axis_name="s",
        num_cores=NC, num_subcores=NS)

    k1 = functools.partial(
        pl.kernel,
        out_type=(jax.ShapeDtypeStruct((D * VP,), jnp.float32),
                  jax.ShapeDtypeStruct((D * VP,), jnp.float32)),
        mesh=mesh,
        compiler_params=pltpu.CompilerParams(use_tc_tiling_on_sc=True),
        scratch_types=[pltpu.VMEM((8, WIN), jnp.float32),
                       pltpu.VMEM((8, WIN), jnp.float32),
                       pltpu.VMEM((8, TAIL), jnp.float32),
                       pltpu.SemaphoreType.DMA,
                       pltpu.SemaphoreType.DMA,
                       pltpu.SemaphoreType.DMA,
                       pltpu.SemaphoreType.DMA],
    )(_k1_body)

    k2 = functools.partial(
        pl.kernel,
        out_type=jax.ShapeDtypeStruct((B,), jnp.float32),
        mesh=mesh,
        compiler_params=pltpu.CompilerParams(use_tc_tiling_on_sc=False),
        scratch_types=[
            pltpu.VMEM((NCH, CH), jnp.int32),
            pltpu.VMEM((NCH, CH), jnp.int32),
            pltpu.VMEM((D * NCH, CH), jnp.int32),
            pltpu.VMEM((D * NCH, CH), jnp.int32),
            pltpu.VMEM((D, BPW), jnp.float32),
            pltpu.VMEM((D, BPW), jnp.float32),
            pltpu.VMEM((BPW,), jnp.float32),
            pltpu.SemaphoreType.DMA,
        ],
    )(_k2_body)

    su, si = k1(embeds_u.T, embeds_i.T)
    iu3 = idx_u.reshape(NW, NCH, CH)
    ii3 = idx_i.reshape(NW, NCH, CH)
    return k2(su, si, iu3, ii3)


# speedup vs baseline: 25.7611x; 1.3118x over previous
import functools

import jax
import jax.numpy as jnp
from jax import lax
from jax.experimental import pallas as pl
from jax.experimental.pallas import tpu as pltpu
from jax.experimental.pallas import tpu_sc as plsc

NC = 2
NS = 16
L = 16
NW = NC * NS

B = 16384
D = 16
V = 1000000
BPW = B // NW
CH = 128
NCH = BPW // CH
NBLK = BPW // L

VP = 1000064
TAIL_OFF = 7812 * 128
TAIL = V - TAIL_OFF

WT = 61
WIN = WT * 128
NWIN = 16
Q2 = 16


def _k1_body(ut_hbm, it_hbm, su_hbm, si_hbm, b0, b1, tail_v, s0, s1, s2, s3):
    wid = lax.axis_index("s") * NC + lax.axis_index("c")
    lane8 = wid % 8
    bufs = (b0, b1)
    semr = (s0, s1)
    semw = (s2, s3)
    start = jnp.where(lane8 < 4, lane8 * 977, 3908 + (lane8 - 4) * 976)

    def window(w, carry):
        for t in range(2):
            src = ut_hbm if t == 0 else it_hbm
            dst = su_hbm if t == 0 else si_hbm
            for s in range(2):
                gid = t * 2 + s
                r8 = s * 8
                for p in range(2):
                    @pl.when((wid // 8 == gid) & (w % 2 == p))
                    def _(src=src, dst=dst, r8=r8, p=p):
                        buf = bufs[p]
                        @pl.when(w >= 2)
                        def _():
                            for rr in range(8):
                                pltpu.make_async_copy(
                                    buf.at[rr, pl.ds(0, WIN)],
                                    dst.at[pl.ds(0, WIN)],
                                    semw[p]).wait()
                        col = (start + w * WT) * 128
                        pltpu.async_copy(
                            src.at[pl.ds(r8, 8), pl.ds(col, WIN)],
                            buf, semr[p]).wait()
                        for rr in range(8):
                            pltpu.async_copy(
                                buf.at[rr, pl.ds(0, WIN)],
                                dst.at[pl.ds((r8 + rr) * VP + col, WIN)],
                                semw[p])
        return carry

    lax.fori_loop(0, NWIN, window, 0)

    for t in range(2):
        dst = su_hbm if t == 0 else si_hbm
        for s in range(2):
            gid = t * 2 + s

            @pl.when(wid // 8 == gid)
            def _(dst=dst):
                for p in range(2):
                    for rr in range(8):
                        pltpu.make_async_copy(
                            bufs[p].at[rr, pl.ds(0, WIN)],
                            dst.at[pl.ds(0, WIN)], semw[p]).wait()

    for t in range(2):
        src = ut_hbm if t == 0 else it_hbm
        dst = su_hbm if t == 0 else si_hbm
        for s in range(2):
            gid = t * 2 + s
            r8 = s * 8

            @pl.when((wid // 8 == gid) & (lane8 < 4))
            def _(src=src, dst=dst, r8=r8):
                col = (start + 976) * 128
                pltpu.async_copy(
                    src.at[pl.ds(r8, 8), pl.ds(col, 128)],
                    b0.at[:, pl.ds(0, 128)], semr[0]).wait()
                for rr in range(8):
                    pltpu.sync_copy(
                        b0.at[rr, pl.ds(0, 128)],
                        dst.at[pl.ds((r8 + rr) * VP + col, 128)])

            @pl.when((wid // 8 == gid) & (lane8 == 7))
            def _(src=src, dst=dst, r8=r8):
                pltpu.sync_copy(
                    src.at[pl.ds(r8, 8), pl.ds(TAIL_OFF, TAIL)], tail_v)
                for rr in range(8):
                    pltpu.sync_copy(
                        tail_v.at[rr],
                        dst.at[pl.ds((r8 + rr) * VP + TAIL_OFF, TAIL)])


def _k2_body(su_hbm, si_hbm, iu_hbm, ii_hbm, out_hbm,
             iu_v, ii_v, ixu_v, ixi_v, gu_v, gi_v, o_v, sem):
    wid = lax.axis_index("s") * NC + lax.axis_index("c")
    base = wid * BPW

    pltpu.sync_copy(iu_hbm.at[wid], iu_v)
    pltpu.sync_copy(ii_hbm.at[wid], ii_v)

    def idxcomp(c, carry):
        for p in range(8):
            ju = iu_v[c, pl.ds(p * 16, 16)]
            ji = ii_v[c, pl.ds(p * 16, 16)]
            for d in range(D):
                ixu_v[d * NCH + c, pl.ds(p * 16, 16)] = ju + d * VP
                ixi_v[d * NCH + c, pl.ds(p * 16, 16)] = ji + d * VP
        return carry

    lax.fori_loop(0, NCH, idxcomp, 0)

    def fire(r, carry):
        d = r // NCH
        c = r % NCH
        pltpu.async_copy(su_hbm.at[ixu_v.at[r]],
                         gu_v.at[d, pl.ds(c * CH, CH)], sem)
        pltpu.async_copy(si_hbm.at[ixi_v.at[r]],
                         gi_v.at[d, pl.ds(c * CH, CH)], sem)

        @pl.when(r >= Q2)
        def _():
            pltpu.make_async_copy(su_hbm.at[ixu_v.at[0]],
                                  gu_v.at[0, pl.ds(0, CH)], sem).wait()
            pltpu.make_async_copy(si_hbm.at[ixi_v.at[0]],
                                  gi_v.at[0, pl.ds(0, CH)], sem).wait()
        return carry

    lax.fori_loop(0, D * NCH, fire, 0)
    for _i in range(Q2):
        pltpu.make_async_copy(su_hbm.at[ixu_v.at[0]],
                              gu_v.at[0, pl.ds(0, CH)], sem).wait()
        pltpu.make_async_copy(si_hbm.at[ixi_v.at[0]],
                              gi_v.at[0, pl.ds(0, CH)], sem).wait()

    def blk(bi, carry):
        kb = bi * L
        acc = gu_v[0, pl.ds(kb, L)] * gi_v[0, pl.ds(kb, L)]
        for d in range(1, D):
            acc = acc + gu_v[d, pl.ds(kb, L)] * gi_v[d, pl.ds(kb, L)]
        o_v[pl.ds(kb, L)] = 1.0 / (1.0 + jnp.exp(-acc))
        return carry

    lax.fori_loop(0, NBLK, blk, 0)

    pltpu.sync_copy(o_v, out_hbm.at[pl.ds(base, BPW)])


def kernel(idx_u, idx_i, embeds_u, embeds_i):
    mesh = plsc.VectorSubcoreMesh(
        core_axis_name="c", subcore_axis_name="s",
        num_cores=NC, num_subcores=NS)

    k1 = functools.partial(
        pl.kernel,
        out_type=(jax.ShapeDtypeStruct((D * VP,), jnp.float32),
                  jax.ShapeDtypeStruct((D * VP,), jnp.float32)),
        mesh=mesh,
        compiler_params=pltpu.CompilerParams(use_tc_tiling_on_sc=True),
        scratch_types=[pltpu.VMEM((8, WIN), jnp.float32),
                       pltpu.VMEM((8, WIN), jnp.float32),
                       pltpu.VMEM((8, TAIL), jnp.float32),
                       pltpu.SemaphoreType.DMA,
                       pltpu.SemaphoreType.DMA,
                       pltpu.SemaphoreType.DMA,
                       pltpu.SemaphoreType.DMA],
    )(_k1_body)

    k2 = functools.partial(
        pl.kernel,
        out_type=jax.ShapeDtypeStruct((B,), jnp.float32),
        mesh=mesh,
        compiler_params=pltpu.CompilerParams(use_tc_tiling_on_sc=False),
        scratch_types=[
            pltpu.VMEM((NCH, CH), jnp.int32),
            pltpu.VMEM((NCH, CH), jnp.int32),
            pltpu.VMEM((D * NCH, CH), jnp.int32),
            pltpu.VMEM((D * NCH, CH), jnp.int32),
            pltpu.VMEM((D, BPW), jnp.float32),
            pltpu.VMEM((D, BPW), jnp.float32),
            pltpu.VMEM((BPW,), jnp.float32),
            pltpu.SemaphoreType.DMA,
        ],
    )(_k2_body)

    su, si = k1(embeds_u.T, embeds_i.T)
    iu3 = idx_u.reshape(NW, NCH, CH)
    ii3 = idx_i.reshape(NW, NCH, CH)
    return k2(su, si, iu3, ii3)


# speedup vs baseline: 26.1272x; 1.0142x over previous
import functools

import jax
import jax.numpy as jnp
from jax import lax
from jax.experimental import pallas as pl
from jax.experimental.pallas import tpu as pltpu
from jax.experimental.pallas import tpu_sc as plsc

NC = 2
NS = 16
L = 16
NW = NC * NS

B = 16384
D = 16
V = 1000000
BPW = B // NW
CH = 128
NCH = BPW // CH
NBLK = BPW // L

VP = 1000064
TAIL_OFF = 7812 * 128
TAIL = V - TAIL_OFF

WT = 61
WIN = WT * 128
NWIN = 16
Q2 = 48


def _k1_body(ut_hbm, it_hbm, su_hbm, si_hbm, b0, b1, tail_v, s0, s1, s2, s3):
    wid = lax.axis_index("s") * NC + lax.axis_index("c")
    lane8 = wid % 8
    bufs = (b0, b1)
    semr = (s0, s1)
    semw = (s2, s3)
    start = jnp.where(lane8 < 4, lane8 * 977, 3908 + (lane8 - 4) * 976)

    def window(w, carry):
        for t in range(2):
            src = ut_hbm if t == 0 else it_hbm
            dst = su_hbm if t == 0 else si_hbm
            for s in range(2):
                gid = t * 2 + s
                r8 = s * 8
                for p in range(2):
                    @pl.when((wid // 8 == gid) & (w % 2 == p))
                    def _(src=src, dst=dst, r8=r8, p=p):
                        buf = bufs[p]
                        @pl.when(w >= 2)
                        def _():
                            for rr in range(8):
                                pltpu.make_async_copy(
                                    buf.at[rr, pl.ds(0, WIN)],
                                    dst.at[pl.ds(0, WIN)],
                                    semw[p]).wait()
                        col = (start + w * WT) * 128
                        pltpu.async_copy(
                            src.at[pl.ds(r8, 8), pl.ds(col, WIN)],
                            buf, semr[p]).wait()
                        for rr in range(8):
                            pltpu.async_copy(
                                buf.at[rr, pl.ds(0, WIN)],
                                dst.at[pl.ds((r8 + rr) * VP + col, WIN)],
                                semw[p])
        return carry

    lax.fori_loop(0, NWIN, window, 0)

    for t in range(2):
        dst = su_hbm if t == 0 else si_hbm
        for s in range(2):
            gid = t * 2 + s

            @pl.when(wid // 8 == gid)
            def _(dst=dst):
                for p in range(2):
                    for rr in range(8):
                        pltpu.make_async_copy(
                            bufs[p].at[rr, pl.ds(0, WIN)],
                            dst.at[pl.ds(0, WIN)], semw[p]).wait()

    for t in range(2):
        src = ut_hbm if t == 0 else it_hbm
        dst = su_hbm if t == 0 else si_hbm
        for s in range(2):
            gid = t * 2 + s
            r8 = s * 8

            @pl.when((wid // 8 == gid) & (lane8 < 4))
            def _(src=src, dst=dst, r8=r8):
                col = (start + 976) * 128
                pltpu.async_copy(
                    src.at[pl.ds(r8, 8), pl.ds(col, 128)],
                    b0.at[:, pl.ds(0, 128)], semr[0]).wait()
                for rr in range(8):
                    pltpu.sync_copy(
                        b0.at[rr, pl.ds(0, 128)],
                        dst.at[pl.ds((r8 + rr) * VP + col, 128)])

            @pl.when((wid // 8 == gid) & (lane8 == 7))
            def _(src=src, dst=dst, r8=r8):
                pltpu.sync_copy(
                    src.at[pl.ds(r8, 8), pl.ds(TAIL_OFF, TAIL)], tail_v)
                for rr in range(8):
                    pltpu.sync_copy(
                        tail_v.at[rr],
                        dst.at[pl.ds((r8 + rr) * VP + TAIL_OFF, TAIL)])


def _k2_body(su_hbm, si_hbm, iu_hbm, ii_hbm, out_hbm,
             iu_v, ii_v, ixu_v, ixi_v, gu_v, gi_v, o_v, sem):
    wid = lax.axis_index("s") * NC + lax.axis_index("c")
    base = wid * BPW

    pltpu.sync_copy(iu_hbm.at[wid], iu_v)
    pltpu.sync_copy(ii_hbm.at[wid], ii_v)

    def idxcomp(c, carry):
        for p in range(8):
            ju = iu_v[c, pl.ds(p * 16, 16)]
            ji = ii_v[c, pl.ds(p * 16, 16)]
            for d in range(D):
                ixu_v[d * NCH + c, pl.ds(p * 16, 16)] = ju + d * VP
                ixi_v[d * NCH + c, pl.ds(p * 16, 16)] = ji + d * VP
        return carry

    lax.fori_loop(0, NCH, idxcomp, 0)

    def fire(r, carry):
        d = r // NCH
        c = r % NCH
        pltpu.async_copy(su_hbm.at[ixu_v.at[r]],
                         gu_v.at[d, pl.ds(c * CH, CH)], sem)
        pltpu.async_copy(si_hbm.at[ixi_v.at[r]],
                         gi_v.at[d, pl.ds(c * CH, CH)], sem)

        @pl.when(r >= Q2)
        def _():
            pltpu.make_async_copy(su_hbm.at[ixu_v.at[0]],
                                  gu_v.at[0, pl.ds(0, CH)], sem).wait()
            pltpu.make_async_copy(si_hbm.at[ixi_v.at[0]],
                                  gi_v.at[0, pl.ds(0, CH)], sem).wait()
        return carry

    lax.fori_loop(0, D * NCH, fire, 0)
    for _i in range(Q2):
        pltpu.make_async_copy(su_hbm.at[ixu_v.at[0]],
                              gu_v.at[0, pl.ds(0, CH)], sem).wait()
        pltpu.make_async_copy(si_hbm.at[ixi_v.at[0]],
                              gi_v.at[0, pl.ds(0, CH)], sem).wait()

    def blk(bi, carry):
        kb = bi * L
        acc = gu_v[0, pl.ds(kb, L)] * gi_v[0, pl.ds(kb, L)]
        for d in range(1, D):
            acc = acc + gu_v[d, pl.ds(kb, L)] * gi_v[d, pl.ds(kb, L)]
        o_v[pl.ds(kb, L)] = 1.0 / (1.0 + jnp.exp(-acc))
        return carry

    lax.fori_loop(0, NBLK, blk, 0)

    pltpu.sync_copy(o_v, out_hbm.at[pl.ds(base, BPW)])


def kernel(idx_u, idx_i, embeds_u, embeds_i):
    mesh = plsc.VectorSubcoreMesh(
        core_axis_name="c", subcore_axis_name="s",
        num_cores=NC, num_subcores=NS)

    k1 = functools.partial(
        pl.kernel,
        out_type=(jax.ShapeDtypeStruct((D * VP,), jnp.float32),
                  jax.ShapeDtypeStruct((D * VP,), jnp.float32)),
        mesh=mesh,
        compiler_params=pltpu.CompilerParams(use_tc_tiling_on_sc=True),
        scratch_types=[pltpu.VMEM((8, WIN), jnp.float32),
                       pltpu.VMEM((8, WIN), jnp.float32),
                       pltpu.VMEM((8, TAIL), jnp.float32),
                       pltpu.SemaphoreType.DMA,
                       pltpu.SemaphoreType.DMA,
                       pltpu.SemaphoreType.DMA,
                       pltpu.SemaphoreType.DMA],
    )(_k1_body)

    k2 = functools.partial(
        pl.kernel,
        out_type=jax.ShapeDtypeStruct((B,), jnp.float32),
        mesh=mesh,
        compiler_params=pltpu.CompilerParams(use_tc_tiling_on_sc=False),
        scratch_types=[
            pltpu.VMEM((NCH, CH), jnp.int32),
            pltpu.VMEM((NCH, CH), jnp.int32),
            pltpu.VMEM((D * NCH, CH), jnp.int32),
            pltpu.VMEM((D * NCH, CH), jnp.int32),
            pltpu.VMEM((D, BPW), jnp.float32),
            pltpu.VMEM((D, BPW), jnp.float32),
            pltpu.VMEM((BPW,), jnp.float32),
            pltpu.SemaphoreType.DMA,
        ],
    )(_k2_body)

    su, si = k1(embeds_u.T, embeds_i.T)
    iu3 = idx_u.reshape(NW, NCH, CH)
    ii3 = idx_i.reshape(NW, NCH, CH)
    return k2(su, si, iu3, ii3)


# speedup vs baseline: 26.2075x; 1.0031x over previous
import functools

import jax
import jax.numpy as jnp
from jax import lax
from jax.experimental import pallas as pl
from jax.experimental.pallas import tpu as pltpu
from jax.experimental.pallas import tpu_sc as plsc

NC = 2
NS = 16
L = 16
NW = NC * NS

B = 16384
D = 16
V = 1000000
BPW = B // NW
CH = 128
NCH = BPW // CH
NBLK = BPW // L

VP = 1000064
TAIL_OFF = 7812 * 128
TAIL = V - TAIL_OFF

WT = 61
WIN = WT * 128
NWIN = 16
Q2 = 64


def _k1_body(ut_hbm, it_hbm, su_hbm, si_hbm, b0, b1, tail_v, s0, s1, s2, s3):
    wid = lax.axis_index("s") * NC + lax.axis_index("c")
    lane8 = wid % 8
    bufs = (b0, b1)
    semr = (s0, s1)
    semw = (s2, s3)
    start = jnp.where(lane8 < 4, lane8 * 977, 3908 + (lane8 - 4) * 976)

    def window(w, carry):
        for t in range(2):
            src = ut_hbm if t == 0 else it_hbm
            dst = su_hbm if t == 0 else si_hbm
            for s in range(2):
                gid = t * 2 + s
                r8 = s * 8
                for p in range(2):
                    @pl.when((wid // 8 == gid) & (w % 2 == p))
                    def _(src=src, dst=dst, r8=r8, p=p):
                        buf = bufs[p]
                        @pl.when(w >= 2)
                        def _():
                            for rr in range(8):
                                pltpu.make_async_copy(
                                    buf.at[rr, pl.ds(0, WIN)],
                                    dst.at[pl.ds(0, WIN)],
                                    semw[p]).wait()
                        col = (start + w * WT) * 128
                        pltpu.async_copy(
                            src.at[pl.ds(r8, 8), pl.ds(col, WIN)],
                            buf, semr[p]).wait()
                        for rr in range(8):
                            pltpu.async_copy(
                                buf.at[rr, pl.ds(0, WIN)],
                                dst.at[pl.ds((r8 + rr) * VP + col, WIN)],
                                semw[p])
        return carry

    lax.fori_loop(0, NWIN, window, 0)

    for t in range(2):
        dst = su_hbm if t == 0 else si_hbm
        for s in range(2):
            gid = t * 2 + s

            @pl.when(wid // 8 == gid)
            def _(dst=dst):
                for p in range(2):
                    for rr in range(8):
                        pltpu.make_async_copy(
                            bufs[p].at[rr, pl.ds(0, WIN)],
                            dst.at[pl.ds(0, WIN)], semw[p]).wait()

    for t in range(2):
        src = ut_hbm if t == 0 else it_hbm
        dst = su_hbm if t == 0 else si_hbm
        for s in range(2):
            gid = t * 2 + s
            r8 = s * 8

            @pl.when((wid // 8 == gid) & (lane8 < 4))
            def _(src=src, dst=dst, r8=r8):
                col = (start + 976) * 128
                pltpu.async_copy(
                    src.at[pl.ds(r8, 8), pl.ds(col, 128)],
                    b0.at[:, pl.ds(0, 128)], semr[0]).wait()
                for rr in range(8):
                    pltpu.sync_copy(
                        b0.at[rr, pl.ds(0, 128)],
                        dst.at[pl.ds((r8 + rr) * VP + col, 128)])

            @pl.when((wid // 8 == gid) & (lane8 == 7))
            def _(src=src, dst=dst, r8=r8):
                pltpu.sync_copy(
                    src.at[pl.ds(r8, 8), pl.ds(TAIL_OFF, TAIL)], tail_v)
                for rr in range(8):
                    pltpu.sync_copy(
                        tail_v.at[rr],
                        dst.at[pl.ds((r8 + rr) * VP + TAIL_OFF, TAIL)])


def _k2_body(su_hbm, si_hbm, iu_hbm, ii_hbm, out_hbm,
             iu_v, ii_v, ixu_v, ixi_v, gu_v, gi_v, o_v, sem):
    wid = lax.axis_index("s") * NC + lax.axis_index("c")
    base = wid * BPW

    pltpu.sync_copy(iu_hbm.at[wid], iu_v)
    pltpu.sync_copy(ii_hbm.at[wid], ii_v)

    def idxcomp(c, carry):
        for p in range(8):
            ju = iu_v[c, pl.ds(p * 16, 16)]
            ji = ii_v[c, pl.ds(p * 16, 16)]
            for d in range(D):
                ixu_v[d * NCH + c, pl.ds(p * 16, 16)] = ju + d * VP
                ixi_v[d * NCH + c, pl.ds(p * 16, 16)] = ji + d * VP
        return carry

    lax.fori_loop(0, NCH, idxcomp, 0)

    def fire(r, carry):
        d = r // NCH
        c = r % NCH
        pltpu.async_copy(su_hbm.at[ixu_v.at[r]],
                         gu_v.at[d, pl.ds(c * CH, CH)], sem)
        pltpu.async_copy(si_hbm.at[ixi_v.at[r]],
                         gi_v.at[d, pl.ds(c * CH, CH)], sem)

        @pl.when(r >= Q2)
        def _():
            pltpu.make_async_copy(su_hbm.at[ixu_v.at[0]],
                                  gu_v.at[0, pl.ds(0, CH)], sem).wait()
            pltpu.make_async_copy(si_hbm.at[ixi_v.at[0]],
                                  gi_v.at[0, pl.ds(0, CH)], sem).wait()
        return carry

    lax.fori_loop(0, D * NCH, fire, 0)
    for _i in range(Q2):
        pltpu.make_async_copy(su_hbm.at[ixu_v.at[0]],
                              gu_v.at[0, pl.ds(0, CH)], sem).wait()
        pltpu.make_async_copy(si_hbm.at[ixi_v.at[0]],
                              gi_v.at[0, pl.ds(0, CH)], sem).wait()

    def blk(bi, carry):
        kb = bi * L
        acc = gu_v[0, pl.ds(kb, L)] * gi_v[0, pl.ds(kb, L)]
        for d in range(1, D):
            acc = acc + gu_v[d, pl.ds(kb, L)] * gi_v[d, pl.ds(kb, L)]
        o_v[pl.ds(kb, L)] = 1.0 / (1.0 + jnp.exp(-acc))
        return carry

    lax.fori_loop(0, NBLK, blk, 0)

    pltpu.sync_copy(o_v, out_hbm.at[pl.ds(base, BPW)])


def kernel(idx_u, idx_i, embeds_u, embeds_i):
    mesh = plsc.VectorSubcoreMesh(
        core_axis_name="c", subcore_axis_name="s",
        num_cores=NC, num_subcores=NS)

    k1 = functools.partial(
        pl.kernel,
        out_type=(jax.ShapeDtypeStruct((D * VP,), jnp.float32),
                  jax.ShapeDtypeStruct((D * VP,), jnp.float32)),
        mesh=mesh,
        compiler_params=pltpu.CompilerParams(use_tc_tiling_on_sc=True),
        scratch_types=[pltpu.VMEM((8, WIN), jnp.float32),
                       pltpu.VMEM((8, WIN), jnp.float32),
                       pltpu.VMEM((8, TAIL), jnp.float32),
                       pltpu.SemaphoreType.DMA,
                       pltpu.SemaphoreType.DMA,
                       pltpu.SemaphoreType.DMA,
                       pltpu.SemaphoreType.DMA],
    )(_k1_body)

    k2 = functools.partial(
        pl.kernel,
        out_type=jax.ShapeDtypeStruct((B,), jnp.float32),
        mesh=mesh,
        compiler_params=pltpu.CompilerParams(use_tc_tiling_on_sc=False),
        scratch_types=[
            pltpu.VMEM((NCH, CH), jnp.int32),
            pltpu.VMEM((NCH, CH), jnp.int32),
            pltpu.VMEM((D * NCH, CH), jnp.int32),
            pltpu.VMEM((D * NCH, CH), jnp.int32),
            pltpu.VMEM((D, BPW), jnp.float32),
            pltpu.VMEM((D, BPW), jnp.float32),
            pltpu.VMEM((BPW,), jnp.float32),
            pltpu.SemaphoreType.DMA,
        ],
    )(_k2_body)

    su, si = k1(embeds_u.T, embeds_i.T)
    iu3 = idx_u.reshape(NW, NCH, CH)
    ii3 = idx_i.reshape(NW, NCH, CH)
    return k2(su, si, iu3, ii3)
